# baseline (speedup 1.0000x reference)
import functools

import jax
import jax.numpy as jnp
from jax import lax
from jax.experimental import pallas as pl
from jax.experimental.pallas import tpu as pltpu
from jax.experimental.pallas import tpu_sc as plsc

_NUM_CORES = 2
_NUM_SUBCORES = 16
_NW = _NUM_CORES * _NUM_SUBCORES
_LANES = 16
_CB = 16
_K = 49


@functools.lru_cache(maxsize=None)
def _emb_lookup(n, d):
    assert d == 16 and n % 128 == 0
    n_cols = n // 128
    per_w = _CB * _K
    span = n_cols - per_w
    cpe = _CB * 128
    half = _CB * 1024
    mesh = plsc.VectorSubcoreMesh(core_axis_name="c", subcore_axis_name="s")

    @functools.partial(
        pl.kernel,
        out_type=jax.ShapeDtypeStruct((n * d,), jnp.float32),
        mesh=mesh,
        scratch_types=(
            [pltpu.VMEM((cpe,), jnp.int32) for _ in range(2)]
            + [pltpu.VMEM((half,), jnp.float32) for _ in range(4)]
            + [pltpu.VMEM((32 * d,), jnp.float32)]
            + [pltpu.SemaphoreType.DMA for _ in range(6)]
        ),
        compiler_params=pltpu.CompilerParams(use_tc_tiling_on_sc=False, needs_layout_passes=False),
    )
    def k(idx_hbm, tblt_hbm, out_hbm, *scratch):
        idx_v = scratch[0:2]
        out_v = (scratch[2:4], scratch[4:6])
        tbl_v = scratch[6]
        sem_a = scratch[7:9]
        sem_c = (scratch[9:11], scratch[11:13])
        wid = lax.axis_index("s") * _NUM_CORES + lax.axis_index("c")
        c_start = wid * span // (_NW - 1)
        pltpu.sync_copy(tblt_hbm, tbl_v)
        tbl_v[pl.ds(0, _LANES)] = jnp.zeros((_LANES,), jnp.float32)

        def a_copy(j, b):
            off = pl.multiple_of((c_start + j * _CB) * 128, 8)
            return pltpu.make_async_copy(
                idx_hbm.at[pl.ds(off, cpe)], idx_v[b], sem_a[b])

        def c_copy(j, b, r):
            off = pl.multiple_of(r * (n_cols * 1024) + (c_start + j * _CB) * 1024, 8)
            return pltpu.make_async_copy(
                out_v[b][r], out_hbm.at[pl.ds(off, half)], sem_c[b][r])

        def compute(b):
            @plsc.parallel_loop(0, _CB * 8, unroll=2)
            def group(g):
                o_base = (g >> 3) * 1024 + (g & 7) * _LANES
                idxs = idx_v[b][pl.ds(g * _LANES, _LANES)] << 4
                for dd in range(d):
                    r, s = dd // 8, dd % 8
                    vals = plsc.load_gather(tbl_v, [idxs + dd])
                    out_v[b][r][pl.ds(o_base + s * 128, _LANES)] = vals

        def step(j, b, do_anext, do_cwait):
            a_copy(j, b).wait()
            if do_anext:
                a_copy(j + 1, 1 - b).start()
            if do_cwait:
                c_copy(j - 2, b, 0).wait()
                c_copy(j - 2, b, 1).wait()
            compute(b)
            c_copy(j, b, 0).start()
            c_copy(j, b, 1).start()

        a_copy(0, 0).start()
        for j in range(3):
            step(j, j % 2, True, j >= 2)

        def body(p, carry):
            step(3 + 2 * p, 1, True, True)
            step(4 + 2 * p, 0, True, True)
            return carry

        n_pairs = (_K - 5) // 2
        lax.fori_loop(0, n_pairs, body, 0)
        for j in range(_K - 2, _K):
            step(j, j % 2, j + 1 < _K, True)
        for j in (_K - 2, _K - 1):
            c_copy(j, j % 2, 0).wait()
            c_copy(j, j % 2, 1).wait()

    return k


def kernel(bond_type, table):
    n = bond_type.shape[0]
    d = table.shape[1]
    flat = _emb_lookup(n, d)(bond_type, table.reshape(-1))
    a = flat.reshape(2, n // 128, 8, 128)
    return a.transpose(1, 3, 0, 2).reshape(n, d)


# speedup vs baseline: 2.8787x; 2.8787x over previous
import functools

import jax
import jax.numpy as jnp
from jax import lax
from jax.experimental import pallas as pl
from jax.experimental.pallas import tpu as pltpu
from jax.experimental.pallas import tpu_sc as plsc

_NUM_CORES = 2
_NUM_SUBCORES = 16
_NW = _NUM_CORES * _NUM_SUBCORES
_LANES = 16
_CB = 16
_K = 49


@functools.lru_cache(maxsize=None)
def _emb_lookup(n, d):
    assert d == 16 and n % 128 == 0
    n_cols = n // 128
    per_w = _CB * _K
    span = n_cols - per_w
    cpe = _CB * 128
    half = _CB * 1024
    mesh = plsc.VectorSubcoreMesh(core_axis_name="c", subcore_axis_name="s")

    @functools.partial(
        pl.kernel,
        out_type=jax.ShapeDtypeStruct((n * d,), jnp.float32),
        mesh=mesh,
        scratch_types=(
            [pltpu.VMEM((cpe,), jnp.int32) for _ in range(2)]
            + [pltpu.VMEM((half,), jnp.float32) for _ in range(4)]
            + [pltpu.VMEM((32 * d,), jnp.float32)]
            + [pltpu.SemaphoreType.DMA for _ in range(6)]
        ),
        compiler_params=pltpu.CompilerParams(use_tc_tiling_on_sc=False, needs_layout_passes=False),
    )
    def k(idx_hbm, tblt_hbm, out_hbm, *scratch):
        idx_v = scratch[0:2]
        out_v = (scratch[2:4], scratch[4:6])
        tbl_v = scratch[6]
        sem_a = scratch[7:9]
        sem_c = (scratch[9:11], scratch[11:13])
        wid = lax.axis_index("s") * _NUM_CORES + lax.axis_index("c")
        c_start = wid * span // (_NW - 1)
        pltpu.sync_copy(tblt_hbm, tbl_v)

        def a_copy(j, b):
            off = pl.multiple_of((c_start + j * _CB) * 128, 8)
            return pltpu.make_async_copy(
                idx_hbm.at[pl.ds(off, cpe)], idx_v[b], sem_a[b])

        def c_copy(j, b, r):
            off = pl.multiple_of(r * (n_cols * 1024) + (c_start + j * _CB) * 1024, 8)
            return pltpu.make_async_copy(
                out_v[b][r], out_hbm.at[pl.ds(off, half)], sem_c[b][r])

        def compute(b):
            @plsc.parallel_loop(0, _CB * 8, unroll=2)
            def group(g):
                o_base = (g >> 3) * 1024 + (g & 7) * _LANES
                idxv = idx_v[b][pl.ds(g * _LANES, _LANES)]
                for dd in range(d):
                    r, s = dd // 8, dd % 8
                    vals = plsc.load_gather(tbl_v, [idxv + (dd * 32)])
                    out_v[b][r][pl.ds(o_base + s * 128, _LANES)] = vals

        def step(j, b, do_anext, do_cwait):
            a_copy(j, b).wait()
            if do_anext:
                a_copy(j + 1, 1 - b).start()
            if do_cwait:
                c_copy(j - 2, b, 0).wait()
                c_copy(j - 2, b, 1).wait()
            compute(b)
            c_copy(j, b, 0).start()
            c_copy(j, b, 1).start()

        a_copy(0, 0).start()
        for j in range(3):
            step(j, j % 2, True, j >= 2)

        def body(p, carry):
            step(3 + 2 * p, 1, True, True)
            step(4 + 2 * p, 0, True, True)
            return carry

        n_pairs = (_K - 5) // 2
        lax.fori_loop(0, n_pairs, body, 0)
        for j in range(_K - 2, _K):
            step(j, j % 2, j + 1 < _K, True)
        for j in (_K - 2, _K - 1):
            c_copy(j, j % 2, 0).wait()
            c_copy(j, j % 2, 1).wait()

    return k


def kernel(bond_type, table):
    n = bond_type.shape[0]
    d = table.shape[1]
    tbl0 = table.at[0].set(jnp.zeros((d,), table.dtype))
    tblt = tbl0.T.reshape(-1)
    flat = _emb_lookup(n, d)(bond_type, tblt)
    a = flat.reshape(2, n // 128, 8, 128)
    return a.transpose(1, 3, 0, 2).reshape(n, d)
